# baseline (speedup 1.0000x reference)
import functools

import jax
import jax.numpy as jnp
from jax import lax
from jax.experimental import pallas as pl
from jax.experimental.pallas import tpu as pltpu
from jax.experimental.pallas import tpu_sc as plsc

PAD = 0
B, S = 4096, 50
D = 64
H2 = 256
N = B * S
EPS = 1e-12

IDX_ROWS = 512
NW = 32
TPW = N // NW
CHUNK = 64
NCHUNK = TPW // CHUNK
DEPTH = 4


def _tri(dtype=jnp.float32):
    r = lax.broadcasted_iota(jnp.int32, (S, S), 0)
    c = lax.broadcasted_iota(jnp.int32, (S, S), 1)
    return (r <= c).astype(dtype)


def _idx_body(src_ref, len_ref, pid2_ref):
    src = src_ref[...]
    m = src != PAD
    mf = m.astype(jnp.float32)
    pid = jnp.dot(mf, _tri(), preferred_element_type=jnp.float32)
    pid = pid.astype(jnp.int32) * m.astype(jnp.int32)
    pid2_ref[...] = jnp.maximum(len_ref[...] + 2 - pid, 0) * m.astype(jnp.int32)


def _make_ids(src, length):
    return pl.pallas_call(
        _idx_body,
        grid=(B // IDX_ROWS,),
        in_specs=[
            pl.BlockSpec((IDX_ROWS, S), lambda i: (i, 0)),
            pl.BlockSpec((IDX_ROWS, 1), lambda i: (i, 0)),
        ],
        out_specs=pl.BlockSpec((IDX_ROWS, S), lambda i: (i, 0)),
        out_shape=jax.ShapeDtypeStruct((B, S), jnp.int32),
    )(src, length.reshape(B, 1))


def _sc_gather_body(pid2_idx, pos2_t, out, idx_all,
                    buf0, buf1, buf2, buf3, bufe0, bufe1, bufe2, bufe3,
                    sg0, sg1, sg2, sg3, so0, so1, so2, so3):
    wid = lax.axis_index("s") * 2 + lax.axis_index("c")
    base = wid * TPW
    bufs = (buf0, buf1, buf2, buf3)
    bufes = (bufe0, bufe1, bufe2, bufe3)
    sem_gs = (sg0, sg1, sg2, sg3)
    sem_os = (so0, so1, so2, so3)

    pltpu.sync_copy(pid2_idx.at[wid], idx_all)
    for p in range(DEPTH):
        pltpu.async_copy(pos2_t.at[idx_all.at[p]], bufs[p], sem_gs[p])

    def ring_body(j, carry):
        for p in range(DEPTH):
            i = DEPTH * j + p
            pltpu.make_async_copy(
                pos2_t.at[pl.ds(0, CHUNK)], bufs[p], sem_gs[p]).wait()

            @pl.when(j >= 1)
            def _():
                pltpu.make_async_copy(
                    bufes[p], out.at[pl.ds(0, CHUNK)], sem_os[p]).wait()

            def compact_body(r4, carry2):
                for rr in range(4):
                    for cc in range(D // 16):
                        r = 4 * r4 + rr
                        bufes[p][r, pl.ds(cc * 16, 16)] = (
                            bufs[p][r, pl.ds(cc * 16, 16)])
                return carry2

            lax.fori_loop(0, CHUNK // 4, compact_body, 0, unroll=False)

            @pl.when(j < (NCHUNK - DEPTH) // DEPTH)
            def _():
                pltpu.async_copy(
                    pos2_t.at[idx_all.at[i + DEPTH]], bufs[p], sem_gs[p])

            pltpu.async_copy(
                bufes[p], out.at[pl.ds(base + i * CHUNK, CHUNK)], sem_os[p])
        return carry

    lax.fori_loop(0, NCHUNK // DEPTH, ring_body, 0, unroll=False)
    for p in range(DEPTH):
        pltpu.make_async_copy(
            bufes[p], out.at[pl.ds(0, CHUNK)], sem_os[p]).wait()


_sc_gather = functools.partial(
    pl.kernel,
    out_type=jax.ShapeDtypeStruct((N, D), jnp.float32),
    mesh=plsc.VectorSubcoreMesh(core_axis_name="c", subcore_axis_name="s"),
    scratch_types=(
        [pltpu.VMEM((NCHUNK, CHUNK), jnp.int32)]
        + [pltpu.VMEM((CHUNK, 2 * D), jnp.float32)] * DEPTH
        + [pltpu.VMEM((CHUNK, D), jnp.float32)] * DEPTH
        + [pltpu.SemaphoreType.DMA] * (2 * DEPTH)
    ),
)(_sc_gather_body)


RB = 64
TOK = RB * S


def _main_body(src_ref, len_ref, e2_ref, aa_ref, pos_ref, g_ref, b_ref,
               w_ref, ub_ref, o_ref):
    src = src_ref[...]
    m = src != PAD
    mf = m.astype(jnp.float32)
    pid = jnp.dot(mf, _tri(), preferred_element_type=jnp.float32)
    pid_i = pid.astype(jnp.int32) * m.astype(jnp.int32)

    oh_pos = (pid_i[..., None] == lax.broadcasted_iota(
        jnp.int32, (RB, S, D), 2)).astype(jnp.float32).reshape(TOK, D)
    oh_aa = (src[..., None] == lax.broadcasted_iota(
        jnp.int32, (RB, S, 32), 2)).astype(jnp.float32).reshape(TOK, 32)
    e = (jnp.dot(oh_pos, pos_ref[...], preferred_element_type=jnp.float32)
         + jnp.dot(oh_aa, aa_ref[...], preferred_element_type=jnp.float32)
         + e2_ref[...])

    mu = jnp.mean(e, axis=1, keepdims=True)
    cent = e - mu
    var = jnp.mean(cent * cent, axis=1, keepdims=True)
    normed = cent * lax.rsqrt(var + EPS) * g_ref[...] + b_ref[...]
    o_ref[...] = (
        jnp.dot(normed, w_ref[...], preferred_element_type=jnp.float32)
        + ub_ref[...]
    ).reshape(RB, S, H2)


def _main(src, length, e2, aa_pad, pos_head, ln_gamma, ln_beta, up_w, up_b):
    return pl.pallas_call(
        _main_body,
        grid=(B // RB,),
        in_specs=[
            pl.BlockSpec((RB, S), lambda i: (i, 0)),
            pl.BlockSpec((RB, 1), lambda i: (i, 0)),
            pl.BlockSpec((TOK, D), lambda i: (i, 0)),
            pl.BlockSpec((32, D), lambda i: (0, 0)),
            pl.BlockSpec((D, D), lambda i: (0, 0)),
            pl.BlockSpec((1, D), lambda i: (0, 0)),
            pl.BlockSpec((1, D), lambda i: (0, 0)),
            pl.BlockSpec((D, H2), lambda i: (0, 0)),
            pl.BlockSpec((1, H2), lambda i: (0, 0)),
        ],
        out_specs=pl.BlockSpec((RB, S, H2), lambda i: (i, 0, 0)),
        out_shape=jax.ShapeDtypeStruct((B, S, H2), jnp.float32),
    )(src, length.reshape(B, 1), e2, aa_pad, pos_head,
      ln_gamma.reshape(1, D), ln_beta.reshape(1, D), up_w,
      up_b.reshape(1, H2))


def kernel(src, length, aa_table, pos_table, pos2_table, ln_gamma, ln_beta,
           up_w, up_b):
    src = src.astype(jnp.int32)
    length = length.astype(jnp.int32)
    pid2 = _make_ids(src, length)
    pos2_pad = jnp.pad(pos2_table, ((0, 0), (0, D)))
    e2 = _sc_gather(pid2.reshape(NW, NCHUNK, CHUNK), pos2_pad)
    aa_pad = jnp.pad(aa_table, ((0, 1), (0, 0)))
    return _main(src, length, e2, aa_pad, pos_table[:D], ln_gamma, ln_beta,
                 up_w, up_b)


# speedup vs baseline: 1.0604x; 1.0604x over previous
import functools

import jax
import jax.numpy as jnp
from jax import lax
from jax.experimental import pallas as pl
from jax.experimental.pallas import tpu as pltpu
from jax.experimental.pallas import tpu_sc as plsc

PAD = 0
B, S = 4096, 50
D = 64
H2 = 256
N = B * S
EPS = 1e-12

IDX_ROWS = 512
NW = 32
TPW = N // NW
CHUNK = 64
NCHUNK = TPW // CHUNK
DEPTH = 4


def _tri(dtype=jnp.float32):
    r = lax.broadcasted_iota(jnp.int32, (S, S), 0)
    c = lax.broadcasted_iota(jnp.int32, (S, S), 1)
    return (r <= c).astype(dtype)


def _idx_body(src_ref, len_ref, pid2_ref):
    src = src_ref[...]
    m = src != PAD
    mf = m.astype(jnp.float32)
    pid = jnp.dot(mf, _tri(), preferred_element_type=jnp.float32)
    pid = pid.astype(jnp.int32) * m.astype(jnp.int32)
    pid2_ref[...] = jnp.maximum(len_ref[...] + 2 - pid, 0) * m.astype(jnp.int32)


def _make_ids(src, length):
    return pl.pallas_call(
        _idx_body,
        grid=(B // IDX_ROWS,),
        in_specs=[
            pl.BlockSpec((IDX_ROWS, S), lambda i: (i, 0)),
            pl.BlockSpec((IDX_ROWS, 1), lambda i: (i, 0)),
        ],
        out_specs=pl.BlockSpec((IDX_ROWS, S), lambda i: (i, 0)),
        out_shape=jax.ShapeDtypeStruct((B, S), jnp.int32),
    )(src, length.reshape(B, 1))


def _sc_gather_body(pid2_idx, pos2_t, out, idx_all,
                    buf0, buf1, buf2, buf3, bufe0, bufe1, bufe2, bufe3,
                    sg0, sg1, sg2, sg3, so0, so1, so2, so3):
    wid = lax.axis_index("s") * 2 + lax.axis_index("c")
    base = wid * TPW
    bufs = (buf0, buf1, buf2, buf3)
    bufes = (bufe0, bufe1, bufe2, bufe3)
    sem_gs = (sg0, sg1, sg2, sg3)
    sem_os = (so0, so1, so2, so3)

    pltpu.sync_copy(pid2_idx.at[wid], idx_all)
    for p in range(DEPTH):
        pltpu.async_copy(pos2_t.at[idx_all.at[p]], bufs[p], sem_gs[p])

    def ring_body(j, carry):
        for p in range(DEPTH):
            i = DEPTH * j + p
            pltpu.make_async_copy(
                pos2_t.at[pl.ds(0, CHUNK)], bufs[p], sem_gs[p]).wait()

            @pl.when(j >= 1)
            def _():
                pltpu.make_async_copy(
                    bufes[p], out.at[pl.ds(0, CHUNK)], sem_os[p]).wait()

            def compact_body(r4, carry2):
                for rr in range(4):
                    for cc in range(D // 16):
                        r = 4 * r4 + rr
                        bufes[p][r, pl.ds(cc * 16, 16)] = (
                            bufs[p][r, pl.ds(cc * 16, 16)])
                return carry2

            lax.fori_loop(0, CHUNK // 4, compact_body, 0, unroll=False)

            @pl.when(j < (NCHUNK - DEPTH) // DEPTH)
            def _():
                pltpu.async_copy(
                    pos2_t.at[idx_all.at[i + DEPTH]], bufs[p], sem_gs[p])

            pltpu.async_copy(
                bufes[p], out.at[pl.ds(base + i * CHUNK, CHUNK)], sem_os[p])
        return carry

    lax.fori_loop(0, NCHUNK // DEPTH, ring_body, 0, unroll=False)
    for p in range(DEPTH):
        pltpu.make_async_copy(
            bufes[p], out.at[pl.ds(0, CHUNK)], sem_os[p]).wait()


_sc_gather = functools.partial(
    pl.kernel,
    out_type=jax.ShapeDtypeStruct((N, D), jnp.float32),
    mesh=plsc.VectorSubcoreMesh(core_axis_name="c", subcore_axis_name="s"),
    scratch_types=(
        [pltpu.VMEM((NCHUNK, CHUNK), jnp.int32)]
        + [pltpu.VMEM((CHUNK, 2 * D), jnp.float32)] * DEPTH
        + [pltpu.VMEM((CHUNK, D), jnp.float32)] * DEPTH
        + [pltpu.SemaphoreType.DMA] * (2 * DEPTH)
    ),
)(_sc_gather_body)


RB = 128
TOK = RB * S


def _main_body(src_ref, len_ref, e2_ref, aa_ref, pos_ref, g_ref, b_ref,
               w_ref, ub_ref, o_ref):
    src = src_ref[...]
    m = src != PAD
    mf = m.astype(jnp.float32)
    pid = jnp.dot(mf, _tri(), preferred_element_type=jnp.float32)
    pid_i = pid.astype(jnp.int32) * m.astype(jnp.int32)

    oh_pos = (pid_i[..., None] == lax.broadcasted_iota(
        jnp.int32, (RB, S, D), 2)).astype(jnp.float32).reshape(TOK, D)
    oh_aa = (src[..., None] == lax.broadcasted_iota(
        jnp.int32, (RB, S, 32), 2)).astype(jnp.float32).reshape(TOK, 32)
    e = (jnp.dot(oh_pos, pos_ref[...], preferred_element_type=jnp.float32)
         + jnp.dot(oh_aa, aa_ref[...], preferred_element_type=jnp.float32)
         + e2_ref[...])

    mu = jnp.mean(e, axis=1, keepdims=True)
    cent = e - mu
    var = jnp.mean(cent * cent, axis=1, keepdims=True)
    normed = cent * lax.rsqrt(var + EPS) * g_ref[...] + b_ref[...]
    o_ref[...] = (
        jnp.dot(normed, w_ref[...], preferred_element_type=jnp.float32)
        + ub_ref[...]
    ).reshape(RB, S, H2)


def _main(src, length, e2, aa_pad, pos_head, ln_gamma, ln_beta, up_w, up_b):
    return pl.pallas_call(
        _main_body,
        grid=(B // RB,),
        in_specs=[
            pl.BlockSpec((RB, S), lambda i: (i, 0)),
            pl.BlockSpec((RB, 1), lambda i: (i, 0)),
            pl.BlockSpec((TOK, D), lambda i: (i, 0)),
            pl.BlockSpec((32, D), lambda i: (0, 0)),
            pl.BlockSpec((D, D), lambda i: (0, 0)),
            pl.BlockSpec((1, D), lambda i: (0, 0)),
            pl.BlockSpec((1, D), lambda i: (0, 0)),
            pl.BlockSpec((D, H2), lambda i: (0, 0)),
            pl.BlockSpec((1, H2), lambda i: (0, 0)),
        ],
        out_specs=pl.BlockSpec((RB, S, H2), lambda i: (i, 0, 0)),
        out_shape=jax.ShapeDtypeStruct((B, S, H2), jnp.float32),
    )(src, length.reshape(B, 1), e2, aa_pad, pos_head,
      ln_gamma.reshape(1, D), ln_beta.reshape(1, D), up_w,
      up_b.reshape(1, H2))


def kernel(src, length, aa_table, pos_table, pos2_table, ln_gamma, ln_beta,
           up_w, up_b):
    src = src.astype(jnp.int32)
    length = length.astype(jnp.int32)
    pid2 = _make_ids(src, length)
    pos2_pad = jnp.pad(pos2_table, ((0, 0), (0, D)))
    e2 = _sc_gather(pid2.reshape(NW, NCHUNK, CHUNK), pos2_pad)
    aa_pad = jnp.pad(aa_table, ((0, 1), (0, 0)))
    return _main(src, length, e2, aa_pad, pos_table[:D], ln_gamma, ln_beta,
                 up_w, up_b)


# speedup vs baseline: 1.0928x; 1.0306x over previous
import functools

import jax
import jax.numpy as jnp
from jax import lax
from jax.experimental import pallas as pl
from jax.experimental.pallas import tpu as pltpu
from jax.experimental.pallas import tpu_sc as plsc

PAD = 0
B, S = 4096, 50
D = 64
H2 = 256
N = B * S
EPS = 1e-12

IDX_ROWS = 512
NW = 32
TPW = N // NW
CHUNK = 64
NCHUNK = TPW // CHUNK
DEPTH = 4


def _tri(dtype=jnp.float32):
    r = lax.broadcasted_iota(jnp.int32, (S, S), 0)
    c = lax.broadcasted_iota(jnp.int32, (S, S), 1)
    return (r <= c).astype(dtype)


def _idx_body(src_ref, len_ref, pid2_ref):
    src = src_ref[...]
    m = src != PAD
    mf = m.astype(jnp.float32)
    pid = jnp.dot(mf, _tri(), preferred_element_type=jnp.float32)
    pid = pid.astype(jnp.int32) * m.astype(jnp.int32)
    pid2_ref[...] = jnp.maximum(len_ref[...] + 2 - pid, 0) * m.astype(jnp.int32)


def _make_ids(src, length):
    return pl.pallas_call(
        _idx_body,
        grid=(B // IDX_ROWS,),
        in_specs=[
            pl.BlockSpec((IDX_ROWS, S), lambda i: (i, 0)),
            pl.BlockSpec((IDX_ROWS, 1), lambda i: (i, 0)),
        ],
        out_specs=pl.BlockSpec((IDX_ROWS, S), lambda i: (i, 0)),
        out_shape=jax.ShapeDtypeStruct((B, S), jnp.int32),
    )(src, length.reshape(B, 1))


def _sc_gather_body(pid2_idx, pos2_t, out, idx_all,
                    buf0, buf1, buf2, buf3, bufe0, bufe1, bufe2, bufe3,
                    sg0, sg1, sg2, sg3, so0, so1, so2, so3):
    wid = lax.axis_index("s") * 2 + lax.axis_index("c")
    base = wid * TPW
    bufs = (buf0, buf1, buf2, buf3)
    bufes = (bufe0, bufe1, bufe2, bufe3)
    sem_gs = (sg0, sg1, sg2, sg3)
    sem_os = (so0, so1, so2, so3)

    pltpu.sync_copy(pid2_idx.at[wid], idx_all)
    for p in range(DEPTH):
        pltpu.async_copy(pos2_t.at[idx_all.at[p]], bufs[p], sem_gs[p])

    def ring_body(j, carry):
        for p in range(DEPTH):
            i = DEPTH * j + p
            pltpu.make_async_copy(
                pos2_t.at[pl.ds(0, CHUNK)], bufs[p], sem_gs[p]).wait()

            @pl.when(j >= 1)
            def _():
                pltpu.make_async_copy(
                    bufes[p], out.at[pl.ds(0, CHUNK)], sem_os[p]).wait()

            def compact_body(r4, carry2):
                for rr in range(4):
                    for cc in range(D // 16):
                        r = 4 * r4 + rr
                        bufes[p][r, pl.ds(cc * 16, 16)] = (
                            bufs[p][r, pl.ds(cc * 16, 16)])
                return carry2

            lax.fori_loop(0, CHUNK // 4, compact_body, 0, unroll=False)

            @pl.when(j < (NCHUNK - DEPTH) // DEPTH)
            def _():
                pltpu.async_copy(
                    pos2_t.at[idx_all.at[i + DEPTH]], bufs[p], sem_gs[p])

            pltpu.async_copy(
                bufes[p], out.at[pl.ds(base + i * CHUNK, CHUNK)], sem_os[p])
        return carry

    lax.fori_loop(0, NCHUNK // DEPTH, ring_body, 0, unroll=False)
    for p in range(DEPTH):
        pltpu.make_async_copy(
            bufes[p], out.at[pl.ds(0, CHUNK)], sem_os[p]).wait()


_sc_gather = functools.partial(
    pl.kernel,
    out_type=jax.ShapeDtypeStruct((N, D), jnp.float32),
    mesh=plsc.VectorSubcoreMesh(core_axis_name="c", subcore_axis_name="s"),
    scratch_types=(
        [pltpu.VMEM((NCHUNK, CHUNK), jnp.int32)]
        + [pltpu.VMEM((CHUNK, 2 * D), jnp.float32)] * DEPTH
        + [pltpu.VMEM((CHUNK, D), jnp.float32)] * DEPTH
        + [pltpu.SemaphoreType.DMA] * (2 * DEPTH)
    ),
)(_sc_gather_body)


RB = 256
TOK = RB * S


def _main_body(src_ref, len_ref, e2_ref, aa_ref, pos_ref, g_ref, b_ref,
               w_ref, ub_ref, o_ref):
    src = src_ref[...]
    m = src != PAD
    mf = m.astype(jnp.float32)
    pid = jnp.dot(mf, _tri(), preferred_element_type=jnp.float32)
    pid_i = pid.astype(jnp.int32) * m.astype(jnp.int32)

    oh_pos = (pid_i[..., None] == lax.broadcasted_iota(
        jnp.int32, (RB, S, D), 2)).astype(jnp.float32).reshape(TOK, D)
    oh_aa = (src[..., None] == lax.broadcasted_iota(
        jnp.int32, (RB, S, 32), 2)).astype(jnp.float32).reshape(TOK, 32)
    e = (jnp.dot(oh_pos, pos_ref[...], preferred_element_type=jnp.float32)
         + jnp.dot(oh_aa, aa_ref[...], preferred_element_type=jnp.float32)
         + e2_ref[...])

    mu = jnp.mean(e, axis=1, keepdims=True)
    cent = e - mu
    var = jnp.mean(cent * cent, axis=1, keepdims=True)
    normed = cent * lax.rsqrt(var + EPS) * g_ref[...] + b_ref[...]
    o_ref[...] = (
        jnp.dot(normed, w_ref[...], preferred_element_type=jnp.float32)
        + ub_ref[...]
    ).reshape(RB, S, H2)


def _main(src, length, e2, aa_pad, pos_head, ln_gamma, ln_beta, up_w, up_b):
    return pl.pallas_call(
        _main_body,
        grid=(B // RB,),
        in_specs=[
            pl.BlockSpec((RB, S), lambda i: (i, 0)),
            pl.BlockSpec((RB, 1), lambda i: (i, 0)),
            pl.BlockSpec((TOK, D), lambda i: (i, 0)),
            pl.BlockSpec((32, D), lambda i: (0, 0)),
            pl.BlockSpec((D, D), lambda i: (0, 0)),
            pl.BlockSpec((1, D), lambda i: (0, 0)),
            pl.BlockSpec((1, D), lambda i: (0, 0)),
            pl.BlockSpec((D, H2), lambda i: (0, 0)),
            pl.BlockSpec((1, H2), lambda i: (0, 0)),
        ],
        out_specs=pl.BlockSpec((RB, S, H2), lambda i: (i, 0, 0)),
        out_shape=jax.ShapeDtypeStruct((B, S, H2), jnp.float32),
    )(src, length.reshape(B, 1), e2, aa_pad, pos_head,
      ln_gamma.reshape(1, D), ln_beta.reshape(1, D), up_w,
      up_b.reshape(1, H2))


def kernel(src, length, aa_table, pos_table, pos2_table, ln_gamma, ln_beta,
           up_w, up_b):
    src = src.astype(jnp.int32)
    length = length.astype(jnp.int32)
    pid2 = _make_ids(src, length)
    pos2_pad = jnp.pad(pos2_table, ((0, 0), (0, D)))
    e2 = _sc_gather(pid2.reshape(NW, NCHUNK, CHUNK), pos2_pad)
    aa_pad = jnp.pad(aa_table, ((0, 1), (0, 0)))
    return _main(src, length, e2, aa_pad, pos_table[:D], ln_gamma, ln_beta,
                 up_w, up_b)
